# Initial kernel scaffold; baseline (speedup 1.0000x reference)
import functools

import jax
import jax.numpy as jnp
from jax import lax
from jax.experimental import pallas as pl
from jax.experimental.pallas import tpu as pltpu
from jax.experimental.pallas import tpu_sc as plsc

VOCAB = 1_000_000
EMB = 32
UNITS = 64
BATCH = 16384
FIELDS = 26

PACK = 4
VP = VOCAB // PACK
BLK = 2500
GRID = VP // BLK

B_TOT = BATCH * FIELDS
NC, NS = 2, 16
NW = NC * NS
PER_W = B_TOT // NW


def _table_body(e4_ref, w1_ref, b1_ref, w2_ref, b2_ref, out_ref):
    e4 = jax.nn.relu(e4_ref[...])
    h1 = jnp.dot(e4, w1_ref[...], preferred_element_type=jnp.float32)
    h1 = jax.nn.relu(h1 + b1_ref[...])
    t4 = jnp.dot(h1, w2_ref[...], preferred_element_type=jnp.float32)
    out_ref[...] = t4 + b2_ref[...]


def _precompute_table(emb4, w1d, b1d, w2d, b2d):
    return pl.pallas_call(
        _table_body,
        grid=(GRID,),
        in_specs=[
            pl.BlockSpec((BLK, EMB * PACK), lambda i: (i, 0)),
            pl.BlockSpec((EMB * PACK, UNITS * PACK), lambda i: (0, 0)),
            pl.BlockSpec((1, UNITS * PACK), lambda i: (0, 0)),
            pl.BlockSpec((UNITS * PACK, PACK), lambda i: (0, 0)),
            pl.BlockSpec((1, 1), lambda i: (0, 0)),
        ],
        out_specs=pl.BlockSpec((BLK, PACK), lambda i: (i, 0)),
        out_shape=jax.ShapeDtypeStruct((VP, PACK), jnp.float32),
    )(emb4, w1d, b1d, w2d, b2d)


@functools.partial(
    pl.kernel,
    out_type=jax.ShapeDtypeStruct((B_TOT,), jnp.float32),
    mesh=plsc.VectorSubcoreMesh(core_axis_name="c", subcore_axis_name="s"),
    scratch_types=[
        pltpu.VMEM((PER_W,), jnp.int32),
        pltpu.VMEM((PER_W,), jnp.float32),
        pltpu.SemaphoreType.DMA,
    ],
)
def _gather_scalars(table_hbm, idx_hbm, out_hbm, idx_v, vals_v, sem):
    wid = lax.axis_index("s") * NC + lax.axis_index("c")
    base = wid * PER_W
    pltpu.sync_copy(idx_hbm.at[pl.ds(base, PER_W)], idx_v)
    pltpu.async_copy(table_hbm.at[idx_v], vals_v, sem).wait()
    pltpu.sync_copy(vals_v, out_hbm.at[pl.ds(base, PER_W)])


def kernel(x, emb, W1, b1, W2, b2):
    eye = jnp.eye(PACK, dtype=jnp.float32)
    w1d = jnp.einsum("ij,ku->ikju", eye, W1).reshape(PACK * EMB, PACK * UNITS)
    w2d = jnp.einsum("ij,uo->iujo", eye, W2).reshape(PACK * UNITS, PACK)
    b1d = jnp.tile(b1, PACK)[None, :]
    b2d = b2.reshape(1, 1)
    emb4 = emb.reshape(VP, PACK * EMB)

    table = _precompute_table(emb4, w1d, b1d, w2d, b2d).reshape(VOCAB)
    idx = x.astype(jnp.int32).reshape(B_TOT)
    out = _gather_scalars(table, idx)
    return out.reshape(BATCH, FIELDS, 1)


# speedup vs baseline: 13.8087x; 13.8087x over previous
import functools

import jax
import jax.numpy as jnp
from jax import lax
from jax.experimental import pallas as pl
from jax.experimental.pallas import tpu as pltpu
from jax.experimental.pallas import tpu_sc as plsc

VOCAB = 1_000_000
EMB = 32
UNITS = 64
BATCH = 16384
FIELDS = 26

PACK = 4
VP = VOCAB // PACK
BLK = 2000
GRID = VP // BLK

B_TOT = BATCH * FIELDS
NC, NS = 2, 16
NW = NC * NS
PER_W = B_TOT // NW


def _table_body(e4_ref, w1_ref, b1_ref, w2_ref, b2_ref, out_ref):
    e4 = jax.nn.relu(e4_ref[...])
    h1 = jnp.dot(e4, w1_ref[...], preferred_element_type=jnp.float32)
    h1 = jax.nn.relu(h1 + b1_ref[...])
    t4 = jnp.dot(h1, w2_ref[...], preferred_element_type=jnp.float32)
    out_ref[...] = t4 + b2_ref[...]


def _precompute_table(emb4, w1d, b1d, w2d, b2d):
    return pl.pallas_call(
        _table_body,
        grid=(GRID,),
        in_specs=[
            pl.BlockSpec((BLK, EMB * PACK), lambda i: (i, 0)),
            pl.BlockSpec((EMB * PACK, UNITS * PACK), lambda i: (0, 0)),
            pl.BlockSpec((1, UNITS * PACK), lambda i: (0, 0)),
            pl.BlockSpec((UNITS * PACK, PACK), lambda i: (0, 0)),
            pl.BlockSpec((1, 1), lambda i: (0, 0)),
        ],
        out_specs=pl.BlockSpec((BLK, PACK), lambda i: (i, 0)),
        out_shape=jax.ShapeDtypeStruct((VP, PACK), jnp.float32),
    )(emb4, w1d, b1d, w2d, b2d)


@functools.partial(
    pl.kernel,
    out_type=jax.ShapeDtypeStruct((B_TOT,), jnp.float32),
    mesh=plsc.VectorSubcoreMesh(core_axis_name="c", subcore_axis_name="s"),
    scratch_types=[
        pltpu.VMEM((PER_W,), jnp.int32),
        pltpu.VMEM((PER_W,), jnp.float32),
        pltpu.SemaphoreType.DMA,
    ],
)
def _gather_scalars(table_hbm, idx_hbm, out_hbm, idx_v, vals_v, sem):
    wid = lax.axis_index("s") * NC + lax.axis_index("c")
    base = wid * PER_W
    pltpu.sync_copy(idx_hbm.at[pl.ds(base, PER_W)], idx_v)
    pltpu.async_copy(table_hbm.at[idx_v], vals_v, sem).wait()
    pltpu.sync_copy(vals_v, out_hbm.at[pl.ds(base, PER_W)])


def kernel(x, emb, W1, b1, W2, b2):
    eye = jnp.eye(PACK, dtype=jnp.float32)
    w1d = jnp.einsum("ij,ku->ikju", eye, W1).reshape(PACK * EMB, PACK * UNITS)
    w2d = jnp.einsum("ij,uo->iujo", eye, W2).reshape(PACK * UNITS, PACK)
    b1d = jnp.tile(b1, PACK)[None, :]
    b2d = b2.reshape(1, 1)
    emb4 = emb.reshape(VP, PACK * EMB)

    table = _precompute_table(emb4, w1d, b1d, w2d, b2d).reshape(VOCAB)
    idx = x.astype(jnp.int32).reshape(B_TOT)
    out = _gather_scalars(table, idx)
    return out.reshape(BATCH, FIELDS, 1)


# speedup vs baseline: 69.0804x; 5.0027x over previous
import functools

import jax
import jax.numpy as jnp
from jax import lax
from jax.experimental import pallas as pl
from jax.experimental.pallas import tpu as pltpu
from jax.experimental.pallas import tpu_sc as plsc

VOCAB = 1_000_000
EMB = 32
UNITS = 64
BATCH = 16384
FIELDS = 26

BL = 8192
GRID = (VOCAB + BL - 1) // BL

B_TOT = BATCH * FIELDS
NC, NS = 2, 16
NW = NC * NS
PER_W = B_TOT // NW


def _table_body(et_ref, w1t_ref, b1_ref, w2t_ref, b2_ref, out_ref):
    e = jax.nn.relu(et_ref[...])
    h1 = jnp.dot(w1t_ref[...], e, preferred_element_type=jnp.float32)
    h1 = jax.nn.relu(h1 + b1_ref[...])
    t = jnp.dot(w2t_ref[...], h1, preferred_element_type=jnp.float32)
    out_ref[...] = (t + b2_ref[...]).reshape(BL)


def _precompute_table(embT, w1t, b1c, w2t, b2c):
    return pl.pallas_call(
        _table_body,
        grid=(GRID,),
        in_specs=[
            pl.BlockSpec((EMB, BL), lambda i: (0, i)),
            pl.BlockSpec((UNITS, EMB), lambda i: (0, 0)),
            pl.BlockSpec((UNITS, 1), lambda i: (0, 0)),
            pl.BlockSpec((1, UNITS), lambda i: (0, 0)),
            pl.BlockSpec((1, 1), lambda i: (0, 0)),
        ],
        out_specs=pl.BlockSpec((BL,), lambda i: (i,)),
        out_shape=jax.ShapeDtypeStruct((VOCAB,), jnp.float32),
    )(embT, w1t, b1c, w2t, b2c)


@functools.partial(
    pl.kernel,
    out_type=jax.ShapeDtypeStruct((B_TOT,), jnp.float32),
    mesh=plsc.VectorSubcoreMesh(core_axis_name="c", subcore_axis_name="s"),
    scratch_types=[
        pltpu.VMEM((PER_W,), jnp.int32),
        pltpu.VMEM((PER_W,), jnp.float32),
        pltpu.SemaphoreType.DMA,
    ],
)
def _gather_scalars(table_hbm, idx_hbm, out_hbm, idx_v, vals_v, sem):
    wid = lax.axis_index("s") * NC + lax.axis_index("c")
    base = wid * PER_W
    pltpu.sync_copy(idx_hbm.at[pl.ds(base, PER_W)], idx_v)
    pltpu.async_copy(table_hbm.at[idx_v], vals_v, sem).wait()
    pltpu.sync_copy(vals_v, out_hbm.at[pl.ds(base, PER_W)])


def kernel(x, emb, W1, b1, W2, b2):
    embT = emb.T
    w1t = W1.T
    b1c = b1[:, None]
    w2t = W2.T
    b2c = b2.reshape(1, 1)

    table = _precompute_table(embT, w1t, b1c, w2t, b2c)
    idx = x.astype(jnp.int32).T.reshape(B_TOT)
    out = _gather_scalars(table, idx)
    return out.reshape(FIELDS, BATCH).T[:, :, None]
